# baseline (speedup 1.0000x reference)
import jax
import jax.numpy as jnp
from jax import lax
from jax.experimental import pallas as pl
from jax.experimental.pallas import tpu as pltpu
from jax.experimental.pallas import tpu_sc as plsc

B = 16384
D = 32
VU = 1_000_000
VQ = 100_000
L = 16
NC, NS = 2, 16
NW = NC * NS
RPW = B // NW
CH = 128
NCH = RPW // CH
GROUPS = RPW // L


def _split_body(ut_ref, qt_ref, *rest):
    uouts = rest[:D]
    qouts = rest[D:2 * D]
    sem = rest[2 * D]
    cps = []
    for c in range(D):
        cp = pltpu.make_async_copy(ut_ref.at[c], uouts[c], sem)
        cp.start()
        cps.append(cp)
        cp = pltpu.make_async_copy(qt_ref.at[c], qouts[c], sem)
        cp.start()
        cps.append(cp)
    for cp in cps:
        cp.wait()


def _split_columns(ut_t, qt_t):
    hbm = pl.BlockSpec(memory_space=pltpu.MemorySpace.HBM)
    return pl.pallas_call(
        _split_body,
        in_specs=[hbm, hbm],
        out_specs=[hbm] * (2 * D),
        out_shape=([jax.ShapeDtypeStruct((VU,), jnp.float32)] * D
                   + [jax.ShapeDtypeStruct((VQ,), jnp.float32)] * D),
        scratch_shapes=[pltpu.SemaphoreType.DMA],
    )(ut_t, qt_t)


def _sc_body(*args):
    uid_hbm, qid_hbm = args[0], args[1]
    ucols = args[2:2 + D]
    qcols = args[2 + D:2 + 2 * D]
    w_hbm = args[2 + 2 * D]
    out_hbm = args[3 + 2 * D]
    iu, iq, rows_ut, rows_qt, wv, outbuf, sem = args[4 + 2 * D:]

    c_ax = lax.axis_index("c")
    s_ax = lax.axis_index("s")
    wid = s_ax * NC + c_ax
    base = wid * RPW

    pltpu.sync_copy(uid_hbm.at[pl.ds(base, RPW)], iu)
    pltpu.sync_copy(qid_hbm.at[pl.ds(base, RPW)], iq)
    pltpu.sync_copy(w_hbm, wv)

    cps = []
    for j in range(NCH):
        sl = pl.ds(j * CH, CH)
        for c in range(D):
            cps.append(pltpu.async_copy(
                ucols[c].at[iu.at[sl]], rows_ut.at[c, sl], sem))
            cps.append(pltpu.async_copy(
                qcols[c].at[iq.at[sl]], rows_qt.at[c, sl], sem))
    for cp in cps:
        cp.wait()

    wvecs = [wv[pl.ds(i * L, L)] for i in range(2 * D // L)]
    dbv = wv[pl.ds(2 * D, L)]
    lanes = lax.iota(jnp.int32, L)
    even = (lanes & 1) == 0
    half = lanes >> 1

    def group(g, carry):
        sl = pl.ds(g * L, L)
        accs = [dbv, jnp.zeros((L,), jnp.float32),
                jnp.zeros((L,), jnp.float32), jnp.zeros((L,), jnp.float32)]
        for c in range(D):
            wu = wvecs[c // L][c % L]
            wq = wvecs[(D + c) // L][(D + c) % L]
            accs[c % 4] = accs[c % 4] + rows_ut[c, sl] * wu
            accs[(c + 1) % 4] = accs[(c + 1) % 4] + rows_qt[c, sl] * wq
        d = (accs[0] + accs[1]) + (accs[2] + accs[3])
        p0 = 1.0 / (1.0 + jnp.exp(-d))
        p1 = 1.0 - p0
        lo0 = p0.at[half].get(mode="promise_in_bounds")
        lo1 = p1.at[half].get(mode="promise_in_bounds")
        hi0 = p0.at[half + 8].get(mode="promise_in_bounds")
        hi1 = p1.at[half + 8].get(mode="promise_in_bounds")
        outbuf[pl.ds(g * 2 * L, L)] = jnp.where(even, lo0, lo1)
        outbuf[pl.ds(g * 2 * L + L, L)] = jnp.where(even, hi0, hi1)
        return carry

    lax.fori_loop(0, GROUPS, group, 0, unroll=False)

    pltpu.sync_copy(outbuf, out_hbm.at[pl.ds(base * 2, RPW * 2)])


@jax.jit
def _run(user_ids, question_ids, ut_t, qt_t, wpk):
    cols = _split_columns(ut_t, qt_t)
    mesh = plsc.VectorSubcoreMesh(core_axis_name="c", subcore_axis_name="s")
    flat = pl.kernel(
        _sc_body,
        mesh=mesh,
        out_type=jax.ShapeDtypeStruct((B * 2,), jnp.float32),
        scratch_types=[
            pltpu.VMEM((RPW,), jnp.int32),
            pltpu.VMEM((RPW,), jnp.int32),
            pltpu.VMEM((D, RPW), jnp.float32),
            pltpu.VMEM((D, RPW), jnp.float32),
            pltpu.VMEM((2 * D + L,), jnp.float32),
            pltpu.VMEM((RPW * 2,), jnp.float32),
            pltpu.SemaphoreType.DMA,
        ],
    )(user_ids, question_ids, *cols, wpk)
    return flat.reshape(B, 2)


def kernel(user_ids, question_ids, user_table, question_table, W, b):
    uid = user_ids.astype(jnp.int32)
    qid = question_ids.astype(jnp.int32)
    wd = W[0] - W[1]
    db = b[0] - b[1]
    wpk = jnp.concatenate([wd, jnp.full((L,), db, jnp.float32)])
    return _run(uid, qid, user_table.T, question_table.T, wpk)


# speedup vs baseline: 1.6548x; 1.6548x over previous
import jax
import jax.numpy as jnp
from jax import lax
from jax.experimental import pallas as pl
from jax.experimental.pallas import tpu as pltpu
from jax.experimental.pallas import tpu_sc as plsc

B = 16384
D = 32
VU = 1_000_000
VQ = 100_000
L = 16
NC, NS = 2, 16
NW = NC * NS
RPW = B // NW
CH = 128
NCH = RPW // CH
GROUPS = RPW // L


def _sc_body(*args):
    uid_hbm, qid_hbm, ut_hbm, qt_hbm, w_hbm, out_hbm = args[:6]
    iu, iq, rows_ut, rows_qt, wv, outbuf, sem = args[6:]

    c_ax = lax.axis_index("c")
    s_ax = lax.axis_index("s")
    wid = s_ax * NC + c_ax
    base = wid * RPW

    pltpu.sync_copy(uid_hbm.at[pl.ds(base, RPW)], iu)
    pltpu.sync_copy(qid_hbm.at[pl.ds(base, RPW)], iq)
    pltpu.sync_copy(w_hbm, wv)

    cps = []
    for j in range(NCH):
        sl = pl.ds(j * CH, CH)
        for c in range(D):
            cps.append(pltpu.async_copy(
                ut_hbm.at[pl.ds(c * VU, VU)].at[iu.at[sl]],
                rows_ut.at[c, sl], sem))
            cps.append(pltpu.async_copy(
                qt_hbm.at[pl.ds(c * VQ, VQ)].at[iq.at[sl]],
                rows_qt.at[c, sl], sem))
    for cp in cps:
        cp.wait()

    wvecs = [wv[pl.ds(i * L, L)] for i in range(2 * D // L)]
    dbv = wv[pl.ds(2 * D, L)]
    lanes = lax.iota(jnp.int32, L)
    even = (lanes & 1) == 0
    half = lanes >> 1

    def group(g, carry):
        sl = pl.ds(g * L, L)
        accs = [dbv, jnp.zeros((L,), jnp.float32),
                jnp.zeros((L,), jnp.float32), jnp.zeros((L,), jnp.float32)]
        for c in range(D):
            wu = wvecs[c // L][c % L]
            wq = wvecs[(D + c) // L][(D + c) % L]
            accs[c % 4] = accs[c % 4] + rows_ut[c, sl] * wu
            accs[(c + 1) % 4] = accs[(c + 1) % 4] + rows_qt[c, sl] * wq
        d = (accs[0] + accs[1]) + (accs[2] + accs[3])
        p0 = 1.0 / (1.0 + jnp.exp(-d))
        p1 = 1.0 - p0
        lo0 = p0.at[half].get(mode="promise_in_bounds")
        lo1 = p1.at[half].get(mode="promise_in_bounds")
        hi0 = p0.at[half + 8].get(mode="promise_in_bounds")
        hi1 = p1.at[half + 8].get(mode="promise_in_bounds")
        outbuf[pl.ds(g * 2 * L, L)] = jnp.where(even, lo0, lo1)
        outbuf[pl.ds(g * 2 * L + L, L)] = jnp.where(even, hi0, hi1)
        return carry

    lax.fori_loop(0, GROUPS, group, 0, unroll=False)

    pltpu.sync_copy(outbuf, out_hbm.at[pl.ds(base * 2, RPW * 2)])


@jax.jit
def _run(user_ids, question_ids, ut, qt, wpk):
    mesh = plsc.VectorSubcoreMesh(core_axis_name="c", subcore_axis_name="s")
    flat = pl.kernel(
        _sc_body,
        mesh=mesh,
        out_type=jax.ShapeDtypeStruct((B * 2,), jnp.float32),
        scratch_types=[
            pltpu.VMEM((RPW,), jnp.int32),
            pltpu.VMEM((RPW,), jnp.int32),
            pltpu.VMEM((D, RPW), jnp.float32),
            pltpu.VMEM((D, RPW), jnp.float32),
            pltpu.VMEM((2 * D + L,), jnp.float32),
            pltpu.VMEM((RPW * 2,), jnp.float32),
            pltpu.SemaphoreType.DMA,
        ],
    )(user_ids, question_ids, ut.T.reshape(D * VU), qt.T.reshape(D * VQ), wpk)
    return flat.reshape(B, 2)


def kernel(user_ids, question_ids, user_table, question_table, W, b):
    uid = user_ids.astype(jnp.int32)
    qid = question_ids.astype(jnp.int32)
    wd = W[0] - W[1]
    db = b[0] - b[1]
    wpk = jnp.concatenate([wd, jnp.full((L,), db, jnp.float32)])
    return _run(uid, qid, user_table, question_table, wpk)


# speedup vs baseline: 7.8070x; 4.7179x over previous
import functools

import jax
import jax.numpy as jnp
from jax import lax
from jax.experimental import pallas as pl
from jax.experimental.pallas import tpu as pltpu
from jax.experimental.pallas import tpu_sc as plsc

B = 16384
D = 32
L = 16
NC, NS = 2, 16
NW = NC * NS
RPW = B // NW
CH = 128
NCH = RPW // CH
GROUPS = RPW // L


def _sc_body(uid_hbm, qid_hbm, ut_hbm, qt_hbm, w_hbm, out_hbm,
             idx_u, idx_q, rows_u, rows_q, wv, outbuf, sem):
    c = lax.axis_index("c")
    s = lax.axis_index("s")
    wid = s * NC + c

    pltpu.sync_copy(uid_hbm.at[pl.ds(wid * NCH, NCH)], idx_u)
    pltpu.sync_copy(qid_hbm.at[pl.ds(wid * NCH, NCH)], idx_q)
    pltpu.sync_copy(w_hbm, wv)

    cps = []
    for j in range(NCH):
        cps.append(pltpu.async_copy(
            ut_hbm.at[idx_u.at[j]], rows_u.at[pl.ds(j * CH, CH)], sem))
        cps.append(pltpu.async_copy(
            qt_hbm.at[idx_q.at[j]], rows_q.at[pl.ds(j * CH, CH)], sem))
    for cp in cps:
        cp.wait()

    wu0 = wv[0, :]
    wu1 = wv[1, :]
    wq0 = wv[2, :]
    wq1 = wv[3, :]
    dbv = wv[4, :]
    lanes = lax.iota(jnp.int32, L)
    masks = [((lanes >> j) & 1) == 1 for j in range(4)]
    perms = [lanes ^ (1 << j) for j in range(4)]

    def group(g, carry):
        vs = []
        for r in range(L):
            row = g * L + r
            vs.append(rows_u[row, pl.ds(0, L)] * wu0
                      + rows_u[row, pl.ds(L, L)] * wu1
                      + rows_q[row, pl.ds(0, L)] * wq0
                      + rows_q[row, pl.ds(L, L)] * wq1)
        j = 0
        while len(vs) > 1:
            nxt = []
            for i in range(len(vs) // 2):
                a, b = vs[2 * i], vs[2 * i + 1]
                ab = jnp.where(masks[j], b, a)
                ba = jnp.where(masks[j], a, b)
                nxt.append(ab + ba.at[perms[j]].get(
                    mode="promise_in_bounds"))
            vs = nxt
            j += 1
        d = vs[0] + dbv
        p0 = 1.0 / (1.0 + jnp.exp(-d))
        p1 = 1.0 - p0
        half = lanes >> 1
        even = (lanes & 1) == 0
        lo0 = p0.at[half].get(mode="promise_in_bounds")
        lo1 = p1.at[half].get(mode="promise_in_bounds")
        hi0 = p0.at[half + 8].get(mode="promise_in_bounds")
        hi1 = p1.at[half + 8].get(mode="promise_in_bounds")
        outbuf[pl.ds(g * 2 * L, L)] = jnp.where(even, lo0, lo1)
        outbuf[pl.ds(g * 2 * L + L, L)] = jnp.where(even, hi0, hi1)
        return carry

    lax.fori_loop(0, GROUPS, group, 0, unroll=False)

    pltpu.sync_copy(outbuf, out_hbm.at[pl.ds(wid * RPW * 2, RPW * 2)])


@jax.jit
def _run(uid2d, qid2d, user_table, question_table, wpk):
    mesh = plsc.VectorSubcoreMesh(core_axis_name="c", subcore_axis_name="s")
    flat = pl.kernel(
        _sc_body,
        mesh=mesh,
        out_type=jax.ShapeDtypeStruct((B * 2,), jnp.float32),
        compiler_params=pltpu.CompilerParams(use_tc_tiling_on_sc=False),
        scratch_types=[
            pltpu.VMEM((NCH, CH), jnp.int32),
            pltpu.VMEM((NCH, CH), jnp.int32),
            pltpu.VMEM((RPW, D), jnp.float32),
            pltpu.VMEM((RPW, D), jnp.float32),
            pltpu.VMEM((5, L), jnp.float32),
            pltpu.VMEM((RPW * 2,), jnp.float32),
            pltpu.SemaphoreType.DMA,
        ],
    )(uid2d, qid2d, user_table, question_table, wpk)
    return flat.reshape(B, 2)


def kernel(user_ids, question_ids, user_table, question_table, W, b):
    uid2d = user_ids.astype(jnp.int32).reshape(NW * NCH, CH)
    qid2d = question_ids.astype(jnp.int32).reshape(NW * NCH, CH)
    wd = W[0] - W[1]
    db = b[0] - b[1]
    wpk = jnp.concatenate([wd, jnp.full((L,), db, jnp.float32)]).reshape(5, L)
    return _run(uid2d, qid2d, user_table, question_table, wpk)
